# baseline (speedup 1.0000x reference)
import jax
import jax.numpy as jnp
import numpy as np
from jax import lax
from jax.experimental import pallas as pl
from jax.experimental.pallas import tpu as pltpu
from jax.experimental.pallas import tpu_sc as plsc

N_INP = 100000
EMBED_DIM = 64
LN_EPS = 1e-5
B, L = 1024, 200
N_ROWS = B * L

NUM_WORKERS = 32
ROWS_PER_WORKER = N_ROWS // NUM_WORKERS
BATCHES_PER_WORKER = B // NUM_WORKERS
ROW_UNROLL = 8

OUT_ROWS, OUT_COLS = 102400, 128
BATCHES_PER_CHUNK = 2
CHUNK_ROWS = BATCHES_PER_CHUNK * L
OUT_ROWS_PER_CHUNK = CHUNK_ROWS * EMBED_DIM // OUT_COLS
CHUNKS_PER_WORKER = BATCHES_PER_WORKER // BATCHES_PER_CHUNK
GPARTS = [(0, 128), (128, 128), (256, 128), (384, 16)]


def _pos_encoding_table(seq_len, channels):
    ch = int(np.ceil(channels / 2) * 2)
    inv_freq = 1.0 / (10000.0 ** (np.arange(0, ch, 2, dtype=np.float32) / ch))
    pos = np.arange(seq_len, dtype=np.float32)
    sin_inp = pos[:, None] * inv_freq[None, :]
    emb = np.stack((np.sin(sin_inp), np.cos(sin_inp)), axis=-1).reshape(seq_len, ch)
    return jnp.asarray(emb[:, :channels], dtype=jnp.float32)


def _rsqrt_vec(t):
    i = plsc.bitcast(t, jnp.int32)
    y = plsc.bitcast(jnp.int32(0x5F3759DF) - (i >> 1), jnp.float32)
    half_t = 0.5 * t
    y = y * (1.5 - half_t * y * y)
    y = y * (1.5 - half_t * y * y)
    return y


def _sc_body(instr_hbm, table_hbm, gamma_hbm, beta_hbm, pe_hbm, out_hbm,
             idx0, idx1, rows0, rows1, out0, out1, pe_v, gamma_v, beta_v,
             si0, si1, sg0, sg1, sw0, sw1):
    wid = lax.axis_index("s") * 2 + lax.axis_index("c")
    base = wid * ROWS_PER_WORKER

    pltpu.sync_copy(pe_hbm, pe_v)
    pltpu.sync_copy(gamma_hbm, gamma_v)
    pltpu.sync_copy(beta_hbm, beta_v)

    idxs = [idx0, idx1]
    rows = [rows0, rows1]
    outs = [out0, out1]
    si = [si0, si1]
    sg = [sg0, sg1]
    sw = [sw0, sw1]

    gvecs = [gamma_v[pl.ds(k * 16, 16)] for k in range(4)]
    bvecs = [beta_v[pl.ds(k * 16, 16)] for k in range(4)]
    inv_d = jnp.float32(1.0 / EMBED_DIM)

    def idx_issue(c, b):
        off = pl.multiple_of(base + c * CHUNK_ROWS, 8)
        pltpu.async_copy(instr_hbm.at[pl.ds(off, CHUNK_ROWS)], idxs[b], si[b])

    def idx_wait(b):
        pltpu.make_async_copy(
            instr_hbm.at[pl.ds(0, CHUNK_ROWS)], idxs[b], si[b]).wait()

    def gather_issue(b):
        for (o, n) in GPARTS:
            pltpu.async_copy(
                table_hbm.at[idxs[b].at[pl.ds(o, n)]],
                rows[b].at[pl.ds(o, n)], sg[b])

    def gather_wait(b):
        for (o, n) in GPARTS:
            pltpu.make_async_copy(
                table_hbm.at[idxs[b].at[pl.ds(o, n)]],
                rows[b].at[pl.ds(o, n)], sg[b]).wait()

    def wout_issue(c, b):
        off = pl.multiple_of((wid * CHUNKS_PER_WORKER + c) * OUT_ROWS_PER_CHUNK, 8)
        pltpu.async_copy(
            outs[b], out_hbm.at[pl.ds(off, OUT_ROWS_PER_CHUNK)], sw[b])

    def wout_wait(b):
        pltpu.make_async_copy(
            outs[b], out_hbm.at[pl.ds(0, OUT_ROWS_PER_CHUNK)], sw[b]).wait()

    perms = [lax.iota(jnp.int32, 16) ^ k for k in (1, 2, 4, 8)]

    def one_row(r, rv, ov):
        lr = r % L
        v = [rv[r, pl.ds(k * 16, 16)] + pe_v[lr, pl.ds(k * 16, 16)]
             for k in range(4)]
        s = (v[0] + v[1]) + (v[2] + v[3])
        q = (v[0] * v[0] + v[1] * v[1]) + (v[2] * v[2] + v[3] * v[3])
        for p in perms:
            s = s + s.at[p].get(mode="promise_in_bounds")
            q = q + q.at[p].get(mode="promise_in_bounds")
        mean = s * inv_d
        t = q * inv_d - mean * mean + LN_EPS
        a = _rsqrt_vec(t)
        cmean = mean * a
        s0 = r // 2
        col = (r % 2) * EMBED_DIM
        for k in range(4):
            ov[s0, pl.ds(col + k * 16, 16)] = \
                (v[k] * a - cmean) * gvecs[k] + bvecs[k]

    def compute(b):
        rv, ov = rows[b], outs[b]

        @plsc.parallel_loop(0, CHUNK_ROWS, 1, unroll=ROW_UNROLL)
        def _(r):
            one_row(r, rv, ov)

    for b in range(2):
        idx_issue(b, b)
    for b in range(2):
        idx_wait(b)
        gather_issue(b)

    for b in range(2):
        gather_wait(b)
        idx_issue(b + 2, b)
        compute(b)
        idx_wait(b)
        gather_issue(b)
        wout_issue(b, b)

    def it_body(it, _):
        for b in range(2):
            c = it * 2 + b
            gather_wait(b)
            idx_issue(c + 2, b)
            wout_wait(b)
            compute(b)
            idx_wait(b)
            gather_issue(b)
            wout_issue(c, b)
        return _

    lax.fori_loop(1, CHUNKS_PER_WORKER // 2 - 1, it_body, None)

    for b in range(2):
        c = CHUNKS_PER_WORKER - 2 + b
        gather_wait(b)
        wout_wait(b)
        compute(b)
        wout_issue(c, b)

    for b in range(2):
        wout_wait(b)


def kernel(instruction, table, ln_gamma, ln_beta):
    instr_flat = instruction.reshape(N_ROWS).astype(jnp.int32)
    pe = _pos_encoding_table(L, EMBED_DIM)

    func = pl.kernel(
        _sc_body,
        out_type=jax.ShapeDtypeStruct((OUT_ROWS, OUT_COLS), jnp.float32),
        mesh=plsc.VectorSubcoreMesh(core_axis_name="c", subcore_axis_name="s"),
        scratch_types=[
            pltpu.VMEM((CHUNK_ROWS,), jnp.int32),
            pltpu.VMEM((CHUNK_ROWS,), jnp.int32),
            pltpu.VMEM((CHUNK_ROWS, EMBED_DIM), jnp.float32),
            pltpu.VMEM((CHUNK_ROWS, EMBED_DIM), jnp.float32),
            pltpu.VMEM((OUT_ROWS_PER_CHUNK, OUT_COLS), jnp.float32),
            pltpu.VMEM((OUT_ROWS_PER_CHUNK, OUT_COLS), jnp.float32),
            pltpu.VMEM((L, EMBED_DIM), jnp.float32),
            pltpu.VMEM((EMBED_DIM,), jnp.float32),
            pltpu.VMEM((EMBED_DIM,), jnp.float32),
            pltpu.SemaphoreType.DMA,
            pltpu.SemaphoreType.DMA,
            pltpu.SemaphoreType.DMA,
            pltpu.SemaphoreType.DMA,
            pltpu.SemaphoreType.DMA,
            pltpu.SemaphoreType.DMA,
        ],
        compiler_params=pltpu.CompilerParams(
            needs_layout_passes=False, use_tc_tiling_on_sc=False),
    )
    out = func(instr_flat, table, ln_gamma, ln_beta, pe)
    return out.reshape(B, L, EMBED_DIM)


# speedup vs baseline: 1.1236x; 1.1236x over previous
import jax
import jax.numpy as jnp
import numpy as np
from jax import lax
from jax.experimental import pallas as pl
from jax.experimental.pallas import tpu as pltpu
from jax.experimental.pallas import tpu_sc as plsc

N_INP = 100000
EMBED_DIM = 64
LN_EPS = 1e-5
B, L = 1024, 200
N_ROWS = B * L

NUM_WORKERS = 32
ROWS_PER_WORKER = N_ROWS // NUM_WORKERS
BATCHES_PER_WORKER = B // NUM_WORKERS
ROW_UNROLL = 4

OUT_ROWS, OUT_COLS = 102400, 128
BATCHES_PER_CHUNK = 2
CHUNK_ROWS = BATCHES_PER_CHUNK * L
OUT_ROWS_PER_CHUNK = CHUNK_ROWS * EMBED_DIM // OUT_COLS
CHUNKS_PER_WORKER = BATCHES_PER_WORKER // BATCHES_PER_CHUNK
GPARTS = [(0, 128), (128, 128), (256, 128), (384, 16)]


def _pos_encoding_table(seq_len, channels):
    ch = int(np.ceil(channels / 2) * 2)
    inv_freq = 1.0 / (10000.0 ** (np.arange(0, ch, 2, dtype=np.float32) / ch))
    pos = np.arange(seq_len, dtype=np.float32)
    sin_inp = pos[:, None] * inv_freq[None, :]
    emb = np.stack((np.sin(sin_inp), np.cos(sin_inp)), axis=-1).reshape(seq_len, ch)
    return jnp.asarray(emb[:, :channels], dtype=jnp.float32)


def _rsqrt_vec(t):
    i = plsc.bitcast(t, jnp.int32)
    y = plsc.bitcast(jnp.int32(0x5F3759DF) - (i >> 1), jnp.float32)
    half_t = 0.5 * t
    y = y * (1.5 - half_t * y * y)
    y = y * (1.5 - half_t * y * y)
    return y


def _sc_body(instr_hbm, table_hbm, gamma_hbm, beta_hbm, pe_hbm, out_hbm,
             idx0, idx1, rows0, rows1, out0, out1, pe_v, gamma_v, beta_v,
             si0, si1, sg0, sg1, sw0, sw1):
    wid = lax.axis_index("s") * 2 + lax.axis_index("c")
    base = wid * ROWS_PER_WORKER

    pltpu.sync_copy(pe_hbm, pe_v)
    pltpu.sync_copy(gamma_hbm, gamma_v)
    pltpu.sync_copy(beta_hbm, beta_v)

    idxs = [idx0, idx1]
    rows = [rows0, rows1]
    outs = [out0, out1]
    si = [si0, si1]
    sg = [sg0, sg1]
    sw = [sw0, sw1]

    gvecs = [gamma_v[pl.ds(k * 16, 16)] for k in range(4)]
    bvecs = [beta_v[pl.ds(k * 16, 16)] for k in range(4)]
    inv_d = jnp.float32(1.0 / EMBED_DIM)

    def idx_issue(c, b):
        off = pl.multiple_of(base + c * CHUNK_ROWS, 8)
        pltpu.async_copy(instr_hbm.at[pl.ds(off, CHUNK_ROWS)], idxs[b], si[b])

    def idx_wait(b):
        pltpu.make_async_copy(
            instr_hbm.at[pl.ds(0, CHUNK_ROWS)], idxs[b], si[b]).wait()

    def gather_issue(b):
        for (o, n) in GPARTS:
            pltpu.async_copy(
                table_hbm.at[idxs[b].at[pl.ds(o, n)]],
                rows[b].at[pl.ds(o, n)], sg[b])

    def gather_wait(b):
        for (o, n) in GPARTS:
            pltpu.make_async_copy(
                table_hbm.at[idxs[b].at[pl.ds(o, n)]],
                rows[b].at[pl.ds(o, n)], sg[b]).wait()

    def wout_issue(c, b):
        off = pl.multiple_of((wid * CHUNKS_PER_WORKER + c) * OUT_ROWS_PER_CHUNK, 8)
        pltpu.async_copy(
            outs[b], out_hbm.at[pl.ds(off, OUT_ROWS_PER_CHUNK)], sw[b])

    def wout_wait(b):
        pltpu.make_async_copy(
            outs[b], out_hbm.at[pl.ds(0, OUT_ROWS_PER_CHUNK)], sw[b]).wait()

    perms = [lax.iota(jnp.int32, 16) ^ k for k in (1, 2, 4, 8)]

    def one_row(r, rv, ov):
        lr = r % L
        v = [rv[r, pl.ds(k * 16, 16)] + pe_v[lr, pl.ds(k * 16, 16)]
             for k in range(4)]
        s = (v[0] + v[1]) + (v[2] + v[3])
        q = (v[0] * v[0] + v[1] * v[1]) + (v[2] * v[2] + v[3] * v[3])
        for p in perms:
            s = s + s.at[p].get(mode="promise_in_bounds")
            q = q + q.at[p].get(mode="promise_in_bounds")
        mean = s * inv_d
        t = q * inv_d - mean * mean + LN_EPS
        a = _rsqrt_vec(t)
        cmean = mean * a
        s0 = r // 2
        col = (r % 2) * EMBED_DIM
        for k in range(4):
            ov[s0, pl.ds(col + k * 16, 16)] = \
                (v[k] * a - cmean) * gvecs[k] + bvecs[k]

    def compute(b):
        rv, ov = rows[b], outs[b]

        @plsc.parallel_loop(0, CHUNK_ROWS, 1, unroll=ROW_UNROLL)
        def _(r):
            one_row(r, rv, ov)

    for b in range(2):
        idx_issue(b, b)
    for b in range(2):
        idx_wait(b)
        gather_issue(b)

    for b in range(2):
        gather_wait(b)
        idx_issue(b + 2, b)
        compute(b)
        idx_wait(b)
        gather_issue(b)
        wout_issue(b, b)

    def it_body(it, _):
        for b in range(2):
            c = it * 2 + b
            gather_wait(b)
            idx_issue(c + 2, b)
            wout_wait(b)
            compute(b)
            idx_wait(b)
            gather_issue(b)
            wout_issue(c, b)
        return _

    lax.fori_loop(1, CHUNKS_PER_WORKER // 2 - 1, it_body, None)

    for b in range(2):
        c = CHUNKS_PER_WORKER - 2 + b
        gather_wait(b)
        wout_wait(b)
        compute(b)
        wout_issue(c, b)

    for b in range(2):
        wout_wait(b)


def kernel(instruction, table, ln_gamma, ln_beta):
    instr_flat = instruction.reshape(N_ROWS).astype(jnp.int32)
    pe = _pos_encoding_table(L, EMBED_DIM)

    func = pl.kernel(
        _sc_body,
        out_type=jax.ShapeDtypeStruct((OUT_ROWS, OUT_COLS), jnp.float32),
        mesh=plsc.VectorSubcoreMesh(core_axis_name="c", subcore_axis_name="s"),
        scratch_types=[
            pltpu.VMEM((CHUNK_ROWS,), jnp.int32),
            pltpu.VMEM((CHUNK_ROWS,), jnp.int32),
            pltpu.VMEM((CHUNK_ROWS, EMBED_DIM), jnp.float32),
            pltpu.VMEM((CHUNK_ROWS, EMBED_DIM), jnp.float32),
            pltpu.VMEM((OUT_ROWS_PER_CHUNK, OUT_COLS), jnp.float32),
            pltpu.VMEM((OUT_ROWS_PER_CHUNK, OUT_COLS), jnp.float32),
            pltpu.VMEM((L, EMBED_DIM), jnp.float32),
            pltpu.VMEM((EMBED_DIM,), jnp.float32),
            pltpu.VMEM((EMBED_DIM,), jnp.float32),
            pltpu.SemaphoreType.DMA,
            pltpu.SemaphoreType.DMA,
            pltpu.SemaphoreType.DMA,
            pltpu.SemaphoreType.DMA,
            pltpu.SemaphoreType.DMA,
            pltpu.SemaphoreType.DMA,
        ],
        compiler_params=pltpu.CompilerParams(
            needs_layout_passes=False, use_tc_tiling_on_sc=False),
    )
    out = func(instr_flat, table, ln_gamma, ln_beta, pe)
    return out.reshape(B, L, EMBED_DIM)
